# Initial kernel scaffold; baseline (speedup 1.0000x reference)
import jax
import jax.numpy as jnp
from jax.experimental import pallas as pl


def kernel(x, edge_index, edge_attr, W1, b1, W2, b2, gamma, beta):
    raise NotImplementedError("write your pallas kernel here")


# speedup vs baseline: 3.2859x; 3.2859x over previous
import functools

import jax
import jax.numpy as jnp
from jax import lax
from jax.experimental import pallas as pl
from jax.experimental.pallas import tpu as pltpu
from jax.experimental.pallas import tpu_sc as plsc

N = 10000
E = 320000
D = 128
DE = 16

NW = 32
BLK = 128
NB = 80
CHUNK = 16
NCH = NB // CHUNK
PW = NB * BLK
EPAD = NW * PW
NPAD = 10240
ROWS_PER_TILE = NPAD // 16


def _sc_scatter_body(zeros_hbm, idx_hbm, attr_hbm, out_hbm, idx_v, attr_v, shared):
    cid = lax.axis_index("c")
    sid = lax.axis_index("s")
    wid = cid * 16 + sid

    row0 = sid * ROWS_PER_TILE
    pltpu.sync_copy(zeros_hbm.at[pl.ds(row0, ROWS_PER_TILE)],
                    shared.at[pl.ds(row0, ROWS_PER_TILE)])
    plsc.subcore_barrier()

    pltpu.sync_copy(idx_hbm.at[wid], idx_v)

    def chunk_body(ch, carry):
        pltpu.sync_copy(attr_hbm.at[wid, pl.ds(ch * CHUNK, CHUNK)], attr_v)
        for b in range(CHUNK):
            pltpu.sync_copy(attr_v.at[b],
                            shared.at[idx_v.at[ch * CHUNK + b]],
                            add=True)
        return carry

    lax.fori_loop(0, NCH, chunk_body, 0)
    plsc.subcore_barrier()

    pltpu.sync_copy(shared.at[pl.ds(row0, ROWS_PER_TILE)],
                    out_hbm.at[cid, pl.ds(row0, ROWS_PER_TILE)])


_sc_scatter = functools.partial(
    pl.kernel,
    out_type=jax.ShapeDtypeStruct((2, NPAD, DE), jnp.float32),
    mesh=plsc.VectorSubcoreMesh(core_axis_name="c", subcore_axis_name="s"),
    scratch_types=[
        pltpu.VMEM((NB, BLK), jnp.int32),
        pltpu.VMEM((CHUNK, BLK, DE), jnp.float32),
        pltpu.VMEM_SHARED((NPAD, DE), jnp.float32),
    ],
    compiler_params=pltpu.CompilerParams(use_tc_tiling_on_sc=False),
)(_sc_scatter_body)


def _tc_mlp_body(x_ref, p0_ref, p1_ref, w1x_ref, w1a_ref, b1_ref, w2_ref,
                 b2_ref, g_ref, bt_ref, o_ref):
    x = x_ref[...]
    agg = p0_ref[...] + p1_ref[...]
    h = (jnp.dot(x, w1x_ref[...], preferred_element_type=jnp.float32)
         + jnp.dot(agg, w1a_ref[...], preferred_element_type=jnp.float32)
         + b1_ref[...])
    h = h * jax.nn.sigmoid(h)
    h = jnp.dot(h, w2_ref[...], preferred_element_type=jnp.float32) + b2_ref[...]
    mu = jnp.mean(h, axis=-1, keepdims=True)
    var = jnp.mean((h - mu) ** 2, axis=-1, keepdims=True)
    h = (h - mu) * lax.rsqrt(var + 1e-5) * g_ref[...] + bt_ref[...]
    o_ref[...] = h + x


def _tc_mlp(x, p0, p1, w1x, w1a, b1, w2, b2, gamma, beta):
    rows = 400
    grid = (N // rows,)
    full = lambda shape: pl.BlockSpec(shape, lambda i: (0, 0))
    return pl.pallas_call(
        _tc_mlp_body,
        grid=grid,
        in_specs=[
            pl.BlockSpec((rows, D), lambda i: (i, 0)),
            pl.BlockSpec((rows, DE), lambda i: (i, 0)),
            pl.BlockSpec((rows, DE), lambda i: (i, 0)),
            full((D, D)),
            full((DE, D)),
            full((1, D)),
            full((D, D)),
            full((1, D)),
            full((1, D)),
            full((1, D)),
        ],
        out_specs=pl.BlockSpec((rows, D), lambda i: (i, 0)),
        out_shape=jax.ShapeDtypeStruct((N, D), jnp.float32),
    )(x, p0, p1, w1x, w1a, b1, w2, b2, gamma, beta)


def kernel(x, edge_index, edge_attr, W1, b1, W2, b2, gamma, beta):
    idx = edge_index[0].astype(jnp.int32)
    pad = EPAD - E
    idx_p = jnp.concatenate([idx, jnp.zeros((pad,), jnp.int32)])
    attr_p = jnp.concatenate(
        [edge_attr, jnp.zeros((pad, DE), edge_attr.dtype)])
    idx_r = idx_p.reshape(NW, NB, BLK)
    attr_r = attr_p.reshape(NW, NB, BLK, DE)
    zeros = jnp.zeros((NPAD, DE), jnp.float32)

    partials = _sc_scatter(zeros, idx_r, attr_r)

    w1x = W1[:D]
    w1a = W1[D:]
    return _tc_mlp(x, partials[0], partials[1], w1x, w1a,
                   b1.reshape(1, D), W2, b2.reshape(1, D),
                   gamma.reshape(1, D), beta.reshape(1, D))


# speedup vs baseline: 5.1963x; 1.5814x over previous
import functools

import jax
import jax.numpy as jnp
from jax import lax
from jax.experimental import pallas as pl
from jax.experimental.pallas import tpu as pltpu
from jax.experimental.pallas import tpu_sc as plsc

N = 10000
E = 320000
D = 128
DE = 16

BLK = 128
NBLK = E // BLK
NWK = 25
BPW = NBLK // NWK
CHUNK = 20
NCH = BPW // CHUNK
EPW = BPW * BLK
ECH = CHUNK * BLK
NPAD = 10240
RPT = NPAD // 16


def _sc_scatter_body(zeros_hbm, idx_hbm, attr_hbm, out_hbm,
                     idx_v, attr_v, shared, lsem, ssem):
    cid = lax.axis_index("c")
    sid = lax.axis_index("s")
    w = sid * 2 + cid
    row0 = sid * RPT

    pltpu.sync_copy(zeros_hbm.at[pl.ds(row0, RPT)],
                    shared.at[pl.ds(row0, RPT)])
    plsc.subcore_barrier()

    @pl.when(w < NWK)
    def _scatter():
        base_b = w * BPW
        base_e = w * EPW
        pltpu.sync_copy(idx_hbm.at[pl.ds(base_b, BPW)], idx_v)

        loads = [None] * NCH
        loads[0] = pltpu.async_copy(
            attr_hbm.at[pl.ds(base_e, ECH)], attr_v.at[0], lsem)
        scatters = [None] * NCH
        for ch in range(NCH):
            buf = ch & 1
            if ch >= 2:
                for d in scatters[ch - 2]:
                    d.wait()
            if ch + 1 < NCH:
                loads[ch + 1] = pltpu.async_copy(
                    attr_hbm.at[pl.ds(base_e + (ch + 1) * ECH, ECH)],
                    attr_v.at[(ch + 1) & 1], lsem)
            loads[ch].wait()
            scatters[ch] = [
                pltpu.async_copy(attr_v.at[buf, pl.ds(b * BLK, BLK)],
                                 shared.at[idx_v.at[ch * CHUNK + b]],
                                 ssem, add=True)
                for b in range(CHUNK)
            ]
        for ch in (NCH - 2, NCH - 1):
            for d in scatters[ch]:
                d.wait()

    plsc.subcore_barrier()

    pltpu.sync_copy(shared.at[pl.ds(row0, RPT)],
                    out_hbm.at[cid, pl.ds(row0, RPT)])


_sc_scatter = functools.partial(
    pl.kernel,
    out_type=jax.ShapeDtypeStruct((2, NPAD, DE), jnp.float32),
    mesh=plsc.VectorSubcoreMesh(core_axis_name="c", subcore_axis_name="s"),
    scratch_types=[
        pltpu.VMEM((BPW, BLK), jnp.int32),
        pltpu.VMEM((2, ECH, DE), jnp.float32),
        pltpu.VMEM_SHARED((NPAD, DE), jnp.float32),
        pltpu.SemaphoreType.DMA,
        pltpu.SemaphoreType.DMA,
    ],
    compiler_params=pltpu.CompilerParams(use_tc_tiling_on_sc=False),
)(_sc_scatter_body)


def _tc_mlp_body(x_ref, p0_ref, p1_ref, w1x_ref, w1a_ref, b1_ref, w2_ref,
                 b2_ref, g_ref, bt_ref, o_ref):
    x = x_ref[...]
    agg = p0_ref[...] + p1_ref[...]
    h = (jnp.dot(x, w1x_ref[...], preferred_element_type=jnp.float32)
         + jnp.dot(agg, w1a_ref[...], preferred_element_type=jnp.float32)
         + b1_ref[...])
    h = h * jax.nn.sigmoid(h)
    h = jnp.dot(h, w2_ref[...], preferred_element_type=jnp.float32) + b2_ref[...]
    mu = jnp.mean(h, axis=-1, keepdims=True)
    var = jnp.mean((h - mu) ** 2, axis=-1, keepdims=True)
    h = (h - mu) * lax.rsqrt(var + 1e-5) * g_ref[...] + bt_ref[...]
    o_ref[...] = h + x


def _tc_mlp(x, p0, p1, w1x, w1a, b1, w2, b2, gamma, beta):
    rows = 400
    grid = (N // rows,)
    full = lambda shape: pl.BlockSpec(shape, lambda i: (0, 0))
    return pl.pallas_call(
        _tc_mlp_body,
        grid=grid,
        in_specs=[
            pl.BlockSpec((rows, D), lambda i: (i, 0)),
            pl.BlockSpec((rows, DE), lambda i: (i, 0)),
            pl.BlockSpec((rows, DE), lambda i: (i, 0)),
            full((D, D)),
            full((DE, D)),
            full((1, D)),
            full((D, D)),
            full((1, D)),
            full((1, D)),
            full((1, D)),
        ],
        out_specs=pl.BlockSpec((rows, D), lambda i: (i, 0)),
        out_shape=jax.ShapeDtypeStruct((N, D), jnp.float32),
    )(x, p0, p1, w1x, w1a, b1, w2, b2, gamma, beta)


def kernel(x, edge_index, edge_attr, W1, b1, W2, b2, gamma, beta):
    idx_r = edge_index[0].astype(jnp.int32).reshape(NBLK, BLK)
    zeros = jnp.zeros((NPAD, DE), jnp.float32)

    partials = _sc_scatter(zeros, idx_r, edge_attr)

    w1x = W1[:D]
    w1a = W1[D:]
    return _tc_mlp(x, partials[0], partials[1], w1x, w1a,
                   b1.reshape(1, D), W2, b2.reshape(1, D),
                   gamma.reshape(1, D), beta.reshape(1, D))
